# baseline (speedup 1.0000x reference)
import functools

import jax
import jax.numpy as jnp
from jax import lax
from jax.experimental import pallas as pl
from jax.experimental.pallas import tpu as pltpu
from jax.experimental.pallas import tpu_sc as plsc

VOCAB = 100000
EMBED = 64
CTX = 20
HIDDEN = 1024
BATCH = 1024

_NC = 2
_NS = 16
_NW = _NC * _NS
_B_FLAT = BATCH * CTX
_B_PER_W = _B_FLAT // _NW

_TN = 2048
_N_TILES = (VOCAB + _TN - 1) // _TN


def _gather_sc(idx_flat, emb_table):
    mesh = plsc.VectorSubcoreMesh(core_axis_name="c", subcore_axis_name="s")

    @functools.partial(
        pl.kernel,
        mesh=mesh,
        out_type=jax.ShapeDtypeStruct((_B_FLAT, EMBED), jnp.float32),
        scratch_types=[
            pltpu.VMEM((_B_PER_W,), jnp.int32),
            pltpu.VMEM((_B_PER_W, EMBED), jnp.float32),
            pltpu.SemaphoreType.DMA,
        ],
        compiler_params=pltpu.CompilerParams(use_tc_tiling_on_sc=False),
    )
    def gather_kernel(idx_hbm, table_hbm, out_hbm, idx_v, rows_v, sem):
        wid = lax.axis_index("s") * _NC + lax.axis_index("c")
        base = wid * _B_PER_W
        pltpu.sync_copy(idx_hbm.at[pl.ds(base, _B_PER_W)], idx_v)
        pltpu.async_copy(table_hbm.at[idx_v], rows_v, sem).wait()
        pltpu.sync_copy(rows_v, out_hbm.at[pl.ds(base, _B_PER_W)])

    return gather_kernel(idx_flat, emb_table)


def _hidden_kernel(emb_ref, w1_ref, b1_ref, h_ref):
    pre = jnp.dot(emb_ref[...], w1_ref[...], preferred_element_type=jnp.float32)
    h_ref[...] = jnp.tanh(pre + b1_ref[...]).astype(jnp.bfloat16)


def _hidden_tc(embeds, W1, b1):
    return pl.pallas_call(
        _hidden_kernel,
        out_shape=jax.ShapeDtypeStruct((BATCH, HIDDEN), jnp.bfloat16),
    )(embeds, W1, b1.reshape(1, HIDDEN))


def _logits_kernel(h_ref, w2_ref, b2_ref, out_ref):
    acc = jnp.dot(
        h_ref[...],
        w2_ref[...].astype(jnp.bfloat16),
        preferred_element_type=jnp.float32,
    )
    out_ref[...] = acc + b2_ref[...]


def _logits_tc(hidden_bf16, W2, b2):
    return pl.pallas_call(
        _logits_kernel,
        grid=(_N_TILES,),
        in_specs=[
            pl.BlockSpec((BATCH, HIDDEN), lambda j: (0, 0)),
            pl.BlockSpec((HIDDEN, _TN), lambda j: (0, j)),
            pl.BlockSpec((1, _TN), lambda j: (0, j)),
        ],
        out_specs=pl.BlockSpec((BATCH, _TN), lambda j: (0, j)),
        out_shape=jax.ShapeDtypeStruct((BATCH, VOCAB), jnp.float32),
        compiler_params=pltpu.CompilerParams(
            dimension_semantics=("arbitrary",),
        ),
    )(hidden_bf16, W2, b2.reshape(1, VOCAB))


def kernel(x, emb_table, W1, b1, W2, b2):
    idx_flat = x.reshape(-1).astype(jnp.int32)
    rows = _gather_sc(idx_flat, emb_table)
    embeds = rows.reshape(BATCH, CTX * EMBED)
    hidden = _hidden_tc(embeds, W1, b1)
    return _logits_tc(hidden, W2, b2)


# speedup vs baseline: 2.8361x; 2.8361x over previous
import functools

import jax
import jax.numpy as jnp
from jax import lax
from jax.experimental import pallas as pl
from jax.experimental.pallas import tpu as pltpu
from jax.experimental.pallas import tpu_sc as plsc

VOCAB = 100000
EMBED = 64
CTX = 20
HIDDEN = 1024
BATCH = 1024

_NC = 2
_NS = 16
_NW = _NC * _NS
_B_FLAT = BATCH * CTX
_B_PER_W = _B_FLAT // _NW

_TN = 2048
_N_TILES = (VOCAB + _TN - 1) // _TN


def _gather_sc(idx_flat, emb_table):
    mesh = plsc.VectorSubcoreMesh(core_axis_name="c", subcore_axis_name="s")

    @functools.partial(
        pl.kernel,
        mesh=mesh,
        out_type=jax.ShapeDtypeStruct((_B_FLAT, EMBED), jnp.float32),
        scratch_types=[
            pltpu.VMEM((_B_PER_W,), jnp.int32),
            pltpu.VMEM((_B_PER_W, EMBED), jnp.float32),
            pltpu.SemaphoreType.DMA,
        ],
        compiler_params=pltpu.CompilerParams(use_tc_tiling_on_sc=False),
    )
    def gather_kernel(idx_hbm, table_hbm, out_hbm, idx_v, rows_v, sem):
        wid = lax.axis_index("s") * _NC + lax.axis_index("c")
        base = wid * _B_PER_W
        pltpu.sync_copy(idx_hbm.at[pl.ds(base, _B_PER_W)], idx_v)
        pltpu.async_copy(table_hbm.at[idx_v], rows_v, sem).wait()
        pltpu.sync_copy(rows_v, out_hbm.at[pl.ds(base, _B_PER_W)])

    return gather_kernel(idx_flat, emb_table)


def _hidden_kernel(emb_ref, w1_ref, b1_ref, ht_ref):
    pre = jnp.dot(emb_ref[...], w1_ref[...], preferred_element_type=jnp.float32)
    ht_ref[...] = jnp.tanh(pre + b1_ref[...]).T.astype(jnp.bfloat16)


def _hidden_tc(embeds, W1, b1):
    return pl.pallas_call(
        _hidden_kernel,
        out_shape=jax.ShapeDtypeStruct((HIDDEN, BATCH), jnp.bfloat16),
    )(embeds, W1, b1.reshape(1, HIDDEN))


def _logits_kernel(ht_ref, w2t_ref, b2_ref, out_ref):
    acc = jnp.dot(
        w2t_ref[...].astype(jnp.bfloat16),
        ht_ref[...],
        preferred_element_type=jnp.float32,
    )
    out_ref[...] = acc + b2_ref[...].T


def _logits_tc(ht_bf16, W2T, b2):
    return pl.pallas_call(
        _logits_kernel,
        grid=(_N_TILES,),
        in_specs=[
            pl.BlockSpec((HIDDEN, BATCH), lambda j: (0, 0)),
            pl.BlockSpec((_TN, HIDDEN), lambda j: (j, 0)),
            pl.BlockSpec((1, _TN), lambda j: (0, j)),
        ],
        out_specs=pl.BlockSpec((_TN, BATCH), lambda j: (j, 0)),
        out_shape=jax.ShapeDtypeStruct((VOCAB, BATCH), jnp.float32),
        compiler_params=pltpu.CompilerParams(
            dimension_semantics=("arbitrary",),
        ),
    )(ht_bf16, W2T, b2.reshape(1, VOCAB))


def kernel(x, emb_table, W1, b1, W2, b2):
    idx_flat = x.reshape(-1).astype(jnp.int32)
    rows = _gather_sc(idx_flat, emb_table)
    embeds = rows.reshape(BATCH, CTX * EMBED)
    ht = _hidden_tc(embeds, W1, b1)
    logits_t = _logits_tc(ht, W2.T, b2)
    return logits_t.T


# speedup vs baseline: 2.8396x; 1.0012x over previous
import functools

import jax
import jax.numpy as jnp
from jax import lax
from jax.experimental import pallas as pl
from jax.experimental.pallas import tpu as pltpu
from jax.experimental.pallas import tpu_sc as plsc

VOCAB = 100000
EMBED = 64
CTX = 20
HIDDEN = 1024
BATCH = 1024

_NC = 2
_NS = 16
_NW = _NC * _NS
_B_FLAT = BATCH * CTX
_B_PER_W = _B_FLAT // _NW

_TN = 2048
_N_TILES = (VOCAB + _TN - 1) // _TN


def _gather_sc(idx_flat, emb_table):
    mesh = plsc.VectorSubcoreMesh(core_axis_name="c", subcore_axis_name="s")

    @functools.partial(
        pl.kernel,
        mesh=mesh,
        out_type=jax.ShapeDtypeStruct((_B_FLAT, EMBED), jnp.float32),
        scratch_types=[
            pltpu.VMEM((_B_PER_W,), jnp.int32),
            pltpu.VMEM((_B_PER_W, EMBED), jnp.float32),
            pltpu.SemaphoreType.DMA,
        ],
        compiler_params=pltpu.CompilerParams(use_tc_tiling_on_sc=False),
    )
    def gather_kernel(idx_hbm, table_hbm, out_hbm, idx_v, rows_v, sem):
        wid = lax.axis_index("s") * _NC + lax.axis_index("c")
        base = wid * _B_PER_W
        pltpu.sync_copy(idx_hbm.at[pl.ds(base, _B_PER_W)], idx_v)
        pltpu.async_copy(table_hbm.at[idx_v], rows_v, sem).wait()
        pltpu.sync_copy(rows_v, out_hbm.at[pl.ds(base, _B_PER_W)])

    return gather_kernel(idx_flat, emb_table)


_KT = CTX * EMBED // 128


def _hidden_kernel(emb_ref, w1_ref, b1_ref, ht_ref, acc_ref):
    t = pl.program_id(0)
    a = emb_ref[...].reshape(BATCH, 128)
    w = w1_ref[...].reshape(128, HIDDEN)
    contrib = jnp.dot(a, w, preferred_element_type=jnp.float32)

    @pl.when(t == 0)
    def _():
        acc_ref[...] = contrib

    @pl.when(t > 0)
    def _():
        acc_ref[...] += contrib

    @pl.when(t == _KT - 1)
    def _():
        ht_ref[...] = jnp.tanh(acc_ref[...] + b1_ref[...]).T.astype(jnp.bfloat16)


def _hidden_tc(emb3, W1, b1):
    return pl.pallas_call(
        _hidden_kernel,
        grid=(_KT,),
        in_specs=[
            pl.BlockSpec((BATCH // 8, 8, 128), lambda t: (t, 0, 0)),
            pl.BlockSpec((1, 128, HIDDEN), lambda t: (t, 0, 0)),
            pl.BlockSpec((1, HIDDEN), lambda t: (0, 0)),
        ],
        out_specs=pl.BlockSpec((HIDDEN, BATCH), lambda t: (0, 0)),
        out_shape=jax.ShapeDtypeStruct((HIDDEN, BATCH), jnp.bfloat16),
        scratch_shapes=[pltpu.VMEM((BATCH, HIDDEN), jnp.float32)],
        compiler_params=pltpu.CompilerParams(
            dimension_semantics=("arbitrary",),
        ),
    )(emb3, W1.reshape(_KT, 128, HIDDEN), b1.reshape(1, HIDDEN))


def _logits_kernel(ht_ref, w2t_ref, b2_ref, out_ref):
    acc = jnp.dot(
        w2t_ref[...].astype(jnp.bfloat16),
        ht_ref[...],
        preferred_element_type=jnp.float32,
    )
    out_ref[...] = acc + b2_ref[...].T


def _logits_tc(ht_bf16, W2T, b2):
    return pl.pallas_call(
        _logits_kernel,
        grid=(_N_TILES,),
        in_specs=[
            pl.BlockSpec((HIDDEN, BATCH), lambda j: (0, 0)),
            pl.BlockSpec((_TN, HIDDEN), lambda j: (j, 0)),
            pl.BlockSpec((1, _TN), lambda j: (0, j)),
        ],
        out_specs=pl.BlockSpec((_TN, BATCH), lambda j: (j, 0)),
        out_shape=jax.ShapeDtypeStruct((VOCAB, BATCH), jnp.float32),
        compiler_params=pltpu.CompilerParams(
            dimension_semantics=("arbitrary",),
        ),
    )(ht_bf16, W2T, b2.reshape(1, VOCAB))


def kernel(x, emb_table, W1, b1, W2, b2):
    idx_perm = (
        x.astype(jnp.int32)
        .reshape(128, 8, CTX // 2, 2)
        .transpose(2, 0, 1, 3)
        .reshape(-1)
    )
    rows = _gather_sc(idx_perm, emb_table)
    emb3 = rows.reshape(CTX * EMBED, 8, 128)
    ht = _hidden_tc(emb3, W1, b1)
    logits_t = _logits_tc(ht, W2.T, b2)
    return logits_t.T


# speedup vs baseline: 2.9733x; 1.0471x over previous
import functools

import jax
import jax.numpy as jnp
from jax import lax
from jax.experimental import pallas as pl
from jax.experimental.pallas import tpu as pltpu
from jax.experimental.pallas import tpu_sc as plsc

VOCAB = 100000
EMBED = 64
CTX = 20
HIDDEN = 1024
BATCH = 1024

_NC = 2
_NS = 16
_NW = _NC * _NS
_B_FLAT = BATCH * CTX
_B_PER_W = _B_FLAT // _NW

_TN = 2048
_N_TILES = (VOCAB + _TN - 1) // _TN


_TT = 2560
_TT_N = -(-VOCAB // (2 * _TT))
_V_PAD = 2 * _TT_N * _TT


def _table_transpose_kernel(a_ref, b_ref, out_ref):
    out_ref[...] = jnp.concatenate(
        [a_ref[...].T, b_ref[...].T], axis=1
    )


def _table_transpose_tc(emb_t):
    return pl.pallas_call(
        _table_transpose_kernel,
        grid=(_TT_N,),
        in_specs=[
            pl.BlockSpec((EMBED, _TT), lambda j: (0, 2 * j)),
            pl.BlockSpec((EMBED, _TT), lambda j: (0, 2 * j + 1)),
        ],
        out_specs=pl.BlockSpec((_TT, 2 * EMBED), lambda j: (j, 0)),
        out_shape=jax.ShapeDtypeStruct((_TT_N * _TT, 2 * EMBED), jnp.float32),
        compiler_params=pltpu.CompilerParams(
            dimension_semantics=("parallel",),
        ),
    )(emb_t, emb_t)


def _gather_sc(idx_flat, emb_table):
    mesh = plsc.VectorSubcoreMesh(core_axis_name="c", subcore_axis_name="s")

    @functools.partial(
        pl.kernel,
        mesh=mesh,
        out_type=jax.ShapeDtypeStruct((_B_FLAT, EMBED), jnp.float32),
        scratch_types=[
            pltpu.VMEM((_B_PER_W,), jnp.int32),
            pltpu.VMEM((_B_PER_W, EMBED), jnp.float32),
            pltpu.SemaphoreType.DMA,
        ],
        compiler_params=pltpu.CompilerParams(use_tc_tiling_on_sc=False),
    )
    def gather_kernel(idx_hbm, table_hbm, out_hbm, idx_v, rows_v, sem):
        wid = lax.axis_index("s") * _NC + lax.axis_index("c")
        base = wid * _B_PER_W
        pltpu.sync_copy(idx_hbm.at[pl.ds(base, _B_PER_W)], idx_v)
        pltpu.async_copy(table_hbm.at[idx_v], rows_v, sem).wait()
        pltpu.sync_copy(rows_v, out_hbm.at[pl.ds(base, _B_PER_W)])

    return gather_kernel(idx_flat, emb_table)


_KT = CTX * EMBED // 128


def _hidden_kernel(emb_ref, w1_ref, b1_ref, ht_ref):
    acc = jnp.zeros((BATCH, HIDDEN), jnp.float32)
    for t in range(_KT):
        a = emb_ref[pl.ds(t * 128, 128), :, :].reshape(BATCH, 128)
        w = w1_ref[t, :, :]
        acc += jnp.dot(a, w, preferred_element_type=jnp.float32)
    ht_ref[...] = jnp.tanh(acc + b1_ref[...]).T.astype(jnp.bfloat16)


def _hidden_tc(emb3, W1, b1):
    return pl.pallas_call(
        _hidden_kernel,
        out_shape=jax.ShapeDtypeStruct((HIDDEN, BATCH), jnp.bfloat16),
    )(emb3, W1.reshape(_KT, 128, HIDDEN), b1.reshape(1, HIDDEN))


def _logits_kernel(ht_ref, w2t_ref, b2_ref, out_ref):
    acc = jnp.dot(
        w2t_ref[...].astype(jnp.bfloat16),
        ht_ref[...],
        preferred_element_type=jnp.float32,
    )
    out_ref[...] = acc + b2_ref[...].T


def _logits_tc(ht_bf16, W2T, b2):
    return pl.pallas_call(
        _logits_kernel,
        grid=(_N_TILES,),
        in_specs=[
            pl.BlockSpec((HIDDEN, BATCH), lambda j: (0, 0)),
            pl.BlockSpec((_TN, HIDDEN), lambda j: (j, 0)),
            pl.BlockSpec((1, _TN), lambda j: (0, j)),
        ],
        out_specs=pl.BlockSpec((_TN, BATCH), lambda j: (j, 0)),
        out_shape=jax.ShapeDtypeStruct((VOCAB, BATCH), jnp.float32),
        compiler_params=pltpu.CompilerParams(
            dimension_semantics=("parallel",),
        ),
    )(ht_bf16, W2T, b2.reshape(1, VOCAB))


def kernel(x, emb_table, W1, b1, W2, b2):
    idx_perm = (
        x.astype(jnp.int32)
        .reshape(128, 8, CTX // 2, 2)
        .transpose(2, 0, 1, 3)
        .reshape(-1)
    )
    j = idx_perm // (2 * _TT)
    k = idx_perm - j * (2 * _TT)
    idx_perm = 2 * (j * _TT + k % _TT) + k // _TT
    table2 = _table_transpose_tc(emb_table.T).reshape(_V_PAD, EMBED)
    rows = _gather_sc(idx_perm, table2)
    emb3 = rows.reshape(CTX * EMBED, 8, 128)
    ht = _hidden_tc(emb3, W1, b1)
    logits_t = _logits_tc(ht, W2.T, b2)
    return logits_t.T


# speedup vs baseline: 3.0028x; 1.0099x over previous
import functools

import jax
import jax.numpy as jnp
from jax import lax
from jax.experimental import pallas as pl
from jax.experimental.pallas import tpu as pltpu
from jax.experimental.pallas import tpu_sc as plsc

VOCAB = 100000
EMBED = 64
CTX = 20
HIDDEN = 1024
BATCH = 1024

_NC = 2
_NS = 16
_NW = _NC * _NS
_B_FLAT = BATCH * CTX
_B_PER_W = _B_FLAT // _NW

_TN = 2048
_N_TILES = (VOCAB + _TN - 1) // _TN


_TT = 2560
_TT_N = -(-VOCAB // (2 * _TT))
_V_PAD = 2 * _TT_N * _TT


def _table_transpose_kernel(a_ref, b_ref, out_ref):
    out_ref[...] = jnp.concatenate(
        [a_ref[...].T, b_ref[...].T], axis=1
    )


def _table_transpose_tc(emb_t):
    return pl.pallas_call(
        _table_transpose_kernel,
        grid=(_TT_N,),
        in_specs=[
            pl.BlockSpec((EMBED, _TT), lambda j: (0, 2 * j)),
            pl.BlockSpec((EMBED, _TT), lambda j: (0, 2 * j + 1)),
        ],
        out_specs=pl.BlockSpec((_TT, 2 * EMBED), lambda j: (j, 0)),
        out_shape=jax.ShapeDtypeStruct((_TT_N * _TT, 2 * EMBED), jnp.float32),
        compiler_params=pltpu.CompilerParams(
            dimension_semantics=("parallel",),
        ),
    )(emb_t, emb_t)


def _gather_sc(idx_flat, emb_table):
    mesh = plsc.VectorSubcoreMesh(core_axis_name="c", subcore_axis_name="s")

    @functools.partial(
        pl.kernel,
        mesh=mesh,
        out_type=jax.ShapeDtypeStruct((_B_FLAT, EMBED), jnp.float32),
        scratch_types=[
            pltpu.VMEM((_B_PER_W,), jnp.int32),
            pltpu.VMEM((_B_PER_W, EMBED), jnp.float32),
            pltpu.SemaphoreType.DMA,
        ],
        compiler_params=pltpu.CompilerParams(use_tc_tiling_on_sc=False),
    )
    def gather_kernel(idx_hbm, table_hbm, out_hbm, idx_v, rows_v, sem):
        wid = lax.axis_index("s") * _NC + lax.axis_index("c")
        base = wid * _B_PER_W
        pltpu.sync_copy(idx_hbm.at[pl.ds(base, _B_PER_W)], idx_v)
        pltpu.async_copy(table_hbm.at[idx_v], rows_v, sem).wait()
        pltpu.sync_copy(rows_v, out_hbm.at[pl.ds(base, _B_PER_W)])

    return gather_kernel(idx_flat, emb_table)


_KT = CTX * EMBED // 128


def _mlp_kernel(emb_ref, w1_ref, b1_ref, w2t_ref, b2_ref, out_ref, ht_ref):
    @pl.when(pl.program_id(0) == 0)
    def _():
        acc = jnp.zeros((BATCH, HIDDEN), jnp.float32)
        for t in range(_KT):
            a = emb_ref[pl.ds(t * 128, 128), :, :].reshape(BATCH, 128)
            w = w1_ref[t, :, :]
            acc += jnp.dot(a, w, preferred_element_type=jnp.float32)
        ht_ref[...] = jnp.tanh(acc + b1_ref[...]).T.astype(jnp.bfloat16)

    acc2 = jnp.dot(
        w2t_ref[...].astype(jnp.bfloat16),
        ht_ref[...],
        preferred_element_type=jnp.float32,
    )
    out_ref[...] = acc2 + b2_ref[...].T


def _mlp_tc(emb3, W1, b1, W2T, b2):
    return pl.pallas_call(
        _mlp_kernel,
        grid=(_N_TILES,),
        in_specs=[
            pl.BlockSpec((CTX * EMBED, 8, 128), lambda j: (0, 0, 0)),
            pl.BlockSpec((_KT, 128, HIDDEN), lambda j: (0, 0, 0)),
            pl.BlockSpec((1, HIDDEN), lambda j: (0, 0)),
            pl.BlockSpec((_TN, HIDDEN), lambda j: (j, 0)),
            pl.BlockSpec((1, _TN), lambda j: (0, j)),
        ],
        out_specs=pl.BlockSpec((_TN, BATCH), lambda j: (j, 0)),
        out_shape=jax.ShapeDtypeStruct((VOCAB, BATCH), jnp.float32),
        scratch_shapes=[pltpu.VMEM((HIDDEN, BATCH), jnp.bfloat16)],
        compiler_params=pltpu.CompilerParams(
            dimension_semantics=("arbitrary",),
        ),
    )(emb3, W1.reshape(_KT, 128, HIDDEN), b1.reshape(1, HIDDEN),
      W2T, b2.reshape(1, VOCAB))


def kernel(x, emb_table, W1, b1, W2, b2):
    idx_perm = (
        x.astype(jnp.int32)
        .reshape(128, 8, CTX // 2, 2)
        .transpose(2, 0, 1, 3)
        .reshape(-1)
    )
    j = idx_perm // (2 * _TT)
    k = idx_perm - j * (2 * _TT)
    idx_perm = 2 * (j * _TT + k % _TT) + k // _TT
    table2 = _table_transpose_tc(emb_table.T).reshape(_V_PAD, EMBED)
    rows = _gather_sc(idx_perm, table2)
    emb3 = rows.reshape(CTX * EMBED, 8, 128)
    logits_t = _mlp_tc(emb3, W1, b1, W2.T, b2)
    return logits_t.T


# speedup vs baseline: 3.0929x; 1.0300x over previous
import functools

import jax
import jax.numpy as jnp
from jax import lax
from jax.experimental import pallas as pl
from jax.experimental.pallas import tpu as pltpu
from jax.experimental.pallas import tpu_sc as plsc

VOCAB = 100000
EMBED = 64
CTX = 20
HIDDEN = 1024
BATCH = 1024

_NC = 2
_NS = 16
_NW = _NC * _NS
_B_FLAT = BATCH * CTX
_B_PER_W = _B_FLAT // _NW

_TN = 2048
_N_TILES = (VOCAB + _TN - 1) // _TN


_TT = 2560
_TT_N = -(-VOCAB // (2 * _TT))
_V_PAD = 2 * _TT_N * _TT


def _table_transpose_kernel(a_ref, b_ref, out_ref):
    out_ref[...] = jnp.concatenate(
        [a_ref[...].T, b_ref[...].T], axis=1
    )


def _table_transpose_tc(emb_t):
    return pl.pallas_call(
        _table_transpose_kernel,
        grid=(_TT_N,),
        in_specs=[
            pl.BlockSpec((EMBED, _TT), lambda j: (0, 2 * j)),
            pl.BlockSpec((EMBED, _TT), lambda j: (0, 2 * j + 1)),
        ],
        out_specs=pl.BlockSpec((_TT, 2 * EMBED), lambda j: (j, 0)),
        out_shape=jax.ShapeDtypeStruct((_TT_N * _TT, 2 * EMBED), jnp.float32),
        compiler_params=pltpu.CompilerParams(
            dimension_semantics=("parallel",),
        ),
    )(emb_t, emb_t)


def _gather_sc(x_cflat, emb_table):
    mesh = plsc.VectorSubcoreMesh(core_axis_name="c", subcore_axis_name="s")

    @functools.partial(
        pl.kernel,
        mesh=mesh,
        out_type=jax.ShapeDtypeStruct((_B_FLAT, EMBED), jnp.float32),
        scratch_types=[
            pltpu.VMEM((_B_FLAT,), jnp.int32),
            pltpu.VMEM((_B_PER_W,), jnp.int32),
            pltpu.VMEM((_B_PER_W, EMBED), jnp.float32),
            pltpu.SemaphoreType.DMA,
        ],
        compiler_params=pltpu.CompilerParams(
            use_tc_tiling_on_sc=False, needs_layout_passes=False
        ),
    )
    def gather_kernel(x_hbm, table_hbm, out_hbm, xall_v, idx_v, rows_v, sem):
        wid = lax.axis_index("s") * _NC + lax.axis_index("c")
        base = wid * _B_PER_W
        pltpu.sync_copy(x_hbm, xall_v)

        def body(qi, carry):
            q = qi * 16
            p = base + q + lax.iota(jnp.int32, 16)
            e = p & 1
            r = (p >> 1) & 7
            m = (p >> 4) & 127
            t = p >> 11
            pos = 2048 * t + 1024 * e + 8 * m + r
            v = plsc.load_gather(xall_v, [pos])
            j2 = v // (2 * _TT)
            k2 = v - j2 * (2 * _TT)
            half = jnp.where(k2 >= _TT, 1, 0).astype(jnp.int32)
            u = 2 * (j2 * _TT + k2 - half * _TT) + half
            idx_v[pl.ds(q, 16)] = u
            return carry

        lax.fori_loop(0, _B_PER_W // 16, body, 0)
        pltpu.async_copy(table_hbm.at[idx_v], rows_v, sem).wait()
        pltpu.sync_copy(rows_v, out_hbm.at[pl.ds(base, _B_PER_W)])

    return gather_kernel(x_cflat, emb_table)


_KT = CTX * EMBED // 128


def _mlp_kernel(emb_ref, w1_ref, b1_ref, w2t_ref, b2_ref, out_ref, ht_ref):
    @pl.when(pl.program_id(0) == 0)
    def _():
        acc = jnp.zeros((BATCH, HIDDEN), jnp.float32)
        for t in range(_KT):
            a = emb_ref[pl.ds(t * 128, 128), :, :].reshape(BATCH, 128)
            w = w1_ref[t, :, :]
            acc += jnp.dot(a, w, preferred_element_type=jnp.float32)
        ht_ref[...] = jnp.tanh(acc + b1_ref[...]).T.astype(jnp.bfloat16)

    acc2 = jnp.dot(
        w2t_ref[...].astype(jnp.bfloat16),
        ht_ref[...],
        preferred_element_type=jnp.float32,
    )
    out_ref[...] = acc2 + b2_ref[...].T


def _mlp_tc(emb3, W1, b1, W2T, b2):
    return pl.pallas_call(
        _mlp_kernel,
        grid=(_N_TILES,),
        in_specs=[
            pl.BlockSpec((CTX * EMBED, 8, 128), lambda j: (0, 0, 0)),
            pl.BlockSpec((_KT, 128, HIDDEN), lambda j: (0, 0, 0)),
            pl.BlockSpec((1, HIDDEN), lambda j: (0, 0)),
            pl.BlockSpec((_TN, HIDDEN), lambda j: (j, 0)),
            pl.BlockSpec((1, _TN), lambda j: (0, j)),
        ],
        out_specs=pl.BlockSpec((_TN, BATCH), lambda j: (j, 0)),
        out_shape=jax.ShapeDtypeStruct((VOCAB, BATCH), jnp.float32),
        scratch_shapes=[pltpu.VMEM((HIDDEN, BATCH), jnp.bfloat16)],
        compiler_params=pltpu.CompilerParams(
            dimension_semantics=("arbitrary",),
        ),
    )(emb3, W1.reshape(_KT, 128, HIDDEN), b1.reshape(1, HIDDEN),
      W2T, b2.reshape(1, VOCAB))


def kernel(x, emb_table, W1, b1, W2, b2):
    x_cflat = x.T.reshape(-1).astype(jnp.int32)
    table2 = _table_transpose_tc(emb_table.T).reshape(_V_PAD, EMBED)
    rows = _gather_sc(x_cflat, table2)
    emb3 = rows.reshape(CTX * EMBED, 8, 128)
    logits_t = _mlp_tc(emb3, W1, b1, W2.T, b2)
    return logits_t.T


# speedup vs baseline: 3.1031x; 1.0033x over previous
import functools

import jax
import jax.numpy as jnp
from jax import lax
from jax.experimental import pallas as pl
from jax.experimental.pallas import tpu as pltpu
from jax.experimental.pallas import tpu_sc as plsc

VOCAB = 100000
EMBED = 64
CTX = 20
HIDDEN = 1024
BATCH = 1024

_NC = 2
_NS = 16
_NW = _NC * _NS
_B_FLAT = BATCH * CTX
_B_PER_W = _B_FLAT // _NW

_TN = 2048
_N_TILES = (VOCAB + _TN - 1) // _TN


_TT = 2560
_TT_N = -(-VOCAB // (2 * _TT))
_V_PAD = 2 * _TT_N * _TT


def _table_transpose_kernel(a_ref, b_ref, out_ref):
    out_ref[...] = jnp.concatenate(
        [a_ref[...].T, b_ref[...].T], axis=1
    )


def _table_transpose_tc(emb_t):
    return pl.pallas_call(
        _table_transpose_kernel,
        grid=(_TT_N,),
        in_specs=[
            pl.BlockSpec((EMBED, _TT), lambda j: (0, 2 * j)),
            pl.BlockSpec((EMBED, _TT), lambda j: (0, 2 * j + 1)),
        ],
        out_specs=pl.BlockSpec((_TT, 2 * EMBED), lambda j: (j, 0)),
        out_shape=jax.ShapeDtypeStruct((_TT_N * _TT, 2 * EMBED), jnp.float32),
        compiler_params=pltpu.CompilerParams(
            dimension_semantics=("parallel",),
        ),
    )(emb_t, emb_t)


def _gather_sc(x_cflat, emb_table):
    mesh = plsc.VectorSubcoreMesh(core_axis_name="c", subcore_axis_name="s")

    @functools.partial(
        pl.kernel,
        mesh=mesh,
        out_type=jax.ShapeDtypeStruct((_B_FLAT, EMBED), jnp.float32),
        scratch_types=[
            pltpu.VMEM((_B_FLAT,), jnp.int32),
            pltpu.VMEM((_B_PER_W,), jnp.int32),
            pltpu.VMEM((_B_PER_W, EMBED), jnp.float32),
            pltpu.SemaphoreType.DMA,
        ],
        compiler_params=pltpu.CompilerParams(
            use_tc_tiling_on_sc=False, needs_layout_passes=False
        ),
    )
    def gather_kernel(x_hbm, table_hbm, out_hbm, xall_v, idx_v, rows_v, sem):
        wid = lax.axis_index("s") * _NC + lax.axis_index("c")
        base = wid * _B_PER_W
        pltpu.sync_copy(x_hbm, xall_v)

        def body(qi, carry):
            q = qi * 16
            p = base + q + lax.iota(jnp.int32, 16)
            e = p & 1
            r = (p >> 1) & 7
            m = (p >> 4) & 127
            t = p >> 11
            pos = 2048 * t + 1024 * e + 8 * m + r
            v = plsc.load_gather(xall_v, [pos])
            j2 = v // (2 * _TT)
            k2 = v - j2 * (2 * _TT)
            half = jnp.where(k2 >= _TT, 1, 0).astype(jnp.int32)
            u = 2 * (j2 * _TT + k2 - half * _TT) + half
            idx_v[pl.ds(q, 16)] = u
            return carry

        lax.fori_loop(0, _B_PER_W // 16, body, 0)
        pltpu.async_copy(table_hbm.at[idx_v], rows_v, sem).wait()
        pltpu.sync_copy(rows_v, out_hbm.at[pl.ds(base, _B_PER_W)])

    return gather_kernel(x_cflat, emb_table)


_KT = CTX * EMBED // 128


def _mlp_kernel(emb_ref, w1_ref, b1_ref, w2t_ref, b2_ref, out_ref, ht_ref):
    @pl.when(pl.program_id(0) == 0)
    def _():
        acc = jnp.zeros((BATCH, HIDDEN), jnp.float32)
        for t in range(_KT):
            a = emb_ref[pl.ds(t * 128, 128), :, :].reshape(BATCH, 128)
            w = w1_ref[t, :, :]
            acc += jnp.dot(a, w, preferred_element_type=jnp.float32)
        ht_ref[...] = jnp.tanh(acc + b1_ref[...]).T

    acc2 = jnp.dot(
        w2t_ref[...],
        ht_ref[...],
        preferred_element_type=jnp.float32,
    )
    out_ref[...] = acc2 + b2_ref[...].T


def _mlp_tc(emb3, W1, b1, W2T, b2):
    return pl.pallas_call(
        _mlp_kernel,
        grid=(_N_TILES,),
        in_specs=[
            pl.BlockSpec((CTX * EMBED, 8, 128), lambda j: (0, 0, 0)),
            pl.BlockSpec((_KT, 128, HIDDEN), lambda j: (0, 0, 0)),
            pl.BlockSpec((1, HIDDEN), lambda j: (0, 0)),
            pl.BlockSpec((_TN, HIDDEN), lambda j: (j, 0)),
            pl.BlockSpec((1, _TN), lambda j: (0, j)),
        ],
        out_specs=pl.BlockSpec((_TN, BATCH), lambda j: (j, 0)),
        out_shape=jax.ShapeDtypeStruct((VOCAB, BATCH), jnp.float32),
        scratch_shapes=[pltpu.VMEM((HIDDEN, BATCH), jnp.float32)],
        compiler_params=pltpu.CompilerParams(
            dimension_semantics=("arbitrary",),
        ),
    )(emb3, W1.reshape(_KT, 128, HIDDEN), b1.reshape(1, HIDDEN),
      W2T, b2.reshape(1, VOCAB))


def kernel(x, emb_table, W1, b1, W2, b2):
    x_cflat = x.T.reshape(-1).astype(jnp.int32)
    table2 = _table_transpose_tc(emb_table.T).reshape(_V_PAD, EMBED)
    rows = _gather_sc(x_cflat, table2)
    emb3 = rows.reshape(CTX * EMBED, 8, 128)
    logits_t = _mlp_tc(emb3, W1, b1, W2.T, b2)
    return logits_t.T


# speedup vs baseline: 3.1402x; 1.0120x over previous
import functools

import jax
import jax.numpy as jnp
from jax import lax
from jax.experimental import pallas as pl
from jax.experimental.pallas import tpu as pltpu
from jax.experimental.pallas import tpu_sc as plsc

VOCAB = 100000
EMBED = 64
CTX = 20
HIDDEN = 1024
BATCH = 1024

_NC = 2
_NS = 16
_NW = _NC * _NS
_B_FLAT = BATCH * CTX
_B_PER_W = _B_FLAT // _NW

_TN = 2048
_N_TILES = (VOCAB + _TN - 1) // _TN


_TT = 5120
_TT_N = -(-VOCAB // (2 * _TT))
_V_PAD = 2 * _TT_N * _TT


def _table_transpose_kernel(a_ref, b_ref, out_ref):
    out_ref[...] = jnp.concatenate(
        [a_ref[...].T, b_ref[...].T], axis=1
    )


def _table_transpose_tc(emb_t):
    return pl.pallas_call(
        _table_transpose_kernel,
        grid=(_TT_N,),
        in_specs=[
            pl.BlockSpec((EMBED, _TT), lambda j: (0, 2 * j)),
            pl.BlockSpec((EMBED, _TT), lambda j: (0, 2 * j + 1)),
        ],
        out_specs=pl.BlockSpec((_TT, 2 * EMBED), lambda j: (j, 0)),
        out_shape=jax.ShapeDtypeStruct((_TT_N * _TT, 2 * EMBED), jnp.float32),
        compiler_params=pltpu.CompilerParams(
            dimension_semantics=("parallel",),
        ),
    )(emb_t, emb_t)


def _gather_sc(x_cflat, emb_table):
    mesh = plsc.VectorSubcoreMesh(core_axis_name="c", subcore_axis_name="s")

    @functools.partial(
        pl.kernel,
        mesh=mesh,
        out_type=jax.ShapeDtypeStruct((_B_FLAT, EMBED), jnp.float32),
        scratch_types=[
            pltpu.VMEM((_B_FLAT,), jnp.int32),
            pltpu.VMEM((_B_PER_W,), jnp.int32),
            pltpu.VMEM((_B_PER_W, EMBED), jnp.float32),
            pltpu.SemaphoreType.DMA,
        ],
        compiler_params=pltpu.CompilerParams(
            use_tc_tiling_on_sc=False, needs_layout_passes=False
        ),
    )
    def gather_kernel(x_hbm, table_hbm, out_hbm, xall_v, idx_v, rows_v, sem):
        wid = lax.axis_index("s") * _NC + lax.axis_index("c")
        base = wid * _B_PER_W
        pltpu.sync_copy(x_hbm, xall_v)

        def body(qi, carry):
            q = qi * 16
            p = base + q + lax.iota(jnp.int32, 16)
            e = p & 1
            r = (p >> 1) & 7
            m = (p >> 4) & 127
            t = p >> 11
            pos = 2048 * t + 1024 * e + 8 * m + r
            v = plsc.load_gather(xall_v, [pos])
            j2 = v // (2 * _TT)
            k2 = v - j2 * (2 * _TT)
            half = jnp.where(k2 >= _TT, 1, 0).astype(jnp.int32)
            u = 2 * (j2 * _TT + k2 - half * _TT) + half
            idx_v[pl.ds(q, 16)] = u
            return carry

        lax.fori_loop(0, _B_PER_W // 16, body, 0)
        pltpu.async_copy(table_hbm.at[idx_v], rows_v, sem).wait()
        pltpu.sync_copy(rows_v, out_hbm.at[pl.ds(base, _B_PER_W)])

    return gather_kernel(x_cflat, emb_table)


_KT = CTX * EMBED // 128


def _mlp_kernel(emb_ref, w1_ref, b1_ref, w2t_ref, b2_ref, out_ref, ht_ref):
    @pl.when(pl.program_id(0) == 0)
    def _():
        acc = jnp.zeros((BATCH, HIDDEN), jnp.float32)
        for t in range(_KT):
            a = emb_ref[pl.ds(t * 128, 128), :, :].reshape(BATCH, 128)
            w = w1_ref[t, :, :]
            acc += jnp.dot(a, w, preferred_element_type=jnp.float32)
        ht_ref[...] = jnp.tanh(acc + b1_ref[...]).T.astype(jnp.bfloat16)

    acc2 = jnp.dot(
        w2t_ref[...].astype(jnp.bfloat16),
        ht_ref[...],
        preferred_element_type=jnp.float32,
    )
    out_ref[...] = acc2 + b2_ref[...].T


def _mlp_tc(emb3, W1, b1, W2T, b2):
    return pl.pallas_call(
        _mlp_kernel,
        grid=(_N_TILES,),
        in_specs=[
            pl.BlockSpec((CTX * EMBED, 8, 128), lambda j: (0, 0, 0)),
            pl.BlockSpec((_KT, 128, HIDDEN), lambda j: (0, 0, 0)),
            pl.BlockSpec((1, HIDDEN), lambda j: (0, 0)),
            pl.BlockSpec((_TN, HIDDEN), lambda j: (j, 0)),
            pl.BlockSpec((1, _TN), lambda j: (0, j)),
        ],
        out_specs=pl.BlockSpec((_TN, BATCH), lambda j: (j, 0)),
        out_shape=jax.ShapeDtypeStruct((VOCAB, BATCH), jnp.float32),
        scratch_shapes=[pltpu.VMEM((HIDDEN, BATCH), jnp.bfloat16)],
        compiler_params=pltpu.CompilerParams(
            dimension_semantics=("arbitrary",),
        ),
    )(emb3, W1.reshape(_KT, 128, HIDDEN), b1.reshape(1, HIDDEN),
      W2T, b2.reshape(1, VOCAB))


def kernel(x, emb_table, W1, b1, W2, b2):
    x_cflat = x.T.reshape(-1).astype(jnp.int32)
    table2 = _table_transpose_tc(emb_table.T).reshape(_V_PAD, EMBED)
    rows = _gather_sc(x_cflat, table2)
    emb3 = rows.reshape(CTX * EMBED, 8, 128)
    logits_t = _mlp_tc(emb3, W1, b1, W2.T, b2)
    return logits_t.T
